# Initial kernel scaffold; baseline (speedup 1.0000x reference)
import jax
import jax.numpy as jnp
from jax.experimental import pallas as pl


def kernel(x, table, W0, b0, W1, b1, W2):
    raise NotImplementedError("write your pallas kernel here")


# speedup vs baseline: 14.6120x; 14.6120x over previous
import functools

import jax
import jax.numpy as jnp
from jax import lax
from jax.experimental import pallas as pl
from jax.experimental.pallas import tpu as pltpu
from jax.experimental.pallas import tpu_sc as plsc

VOCAB = 1000000
EMBED = 16
FIELDS = 26
BATCH = 16384

N = BATCH * FIELDS
NC = 2
NS = 16
NW = NC * NS
PER_W = N // NW
CHUNK = 128
NCHUNK = PER_W // CHUNK


def _make_gather():
    mesh = plsc.VectorSubcoreMesh(core_axis_name="c", subcore_axis_name="s")

    @functools.partial(
        pl.kernel,
        mesh=mesh,
        out_type=jax.ShapeDtypeStruct((N, EMBED), jnp.float32),
        scratch_types=[
            pltpu.VMEM((NCHUNK, CHUNK), jnp.int32),
            pltpu.VMEM((CHUNK, EMBED), jnp.float32),
            pltpu.SemaphoreType.DMA,
        ],
        compiler_params=pltpu.CompilerParams(use_tc_tiling_on_sc=False),
    )
    def gather_k(x_hbm, table_hbm, out_hbm, idx_v, rows_v, sem):
        wid = lax.axis_index("s") * NC + lax.axis_index("c")
        pltpu.sync_copy(x_hbm.at[wid], idx_v)
        base = wid * PER_W

        def body(j, carry):
            pltpu.async_copy(table_hbm.at[idx_v.at[j]], rows_v, sem).wait()
            pltpu.sync_copy(rows_v, out_hbm.at[pl.ds(base + j * CHUNK, CHUNK)])
            return carry

        lax.fori_loop(0, NCHUNK, body, 0)

    return gather_k


_gather = _make_gather()


def _mlp_body(e_ref, w0_ref, b0_ref, w1_ref, b1_ref, w2_ref, o_ref):
    h = jnp.dot(e_ref[...], w0_ref[...], preferred_element_type=jnp.float32)
    h = jnp.maximum(h + b0_ref[...], 0.0)
    h = jnp.dot(h, w1_ref[...], preferred_element_type=jnp.float32)
    h = jnp.maximum(h + b1_ref[...], 0.0)
    logits = jnp.sum(h * w2_ref[...], axis=1, keepdims=True)
    o_ref[...] = 1.0 / (1.0 + jnp.exp(-logits))


def _mlp(embeds, W0, b0, W1, b1, W2):
    BB = 2048
    grid = (BATCH // BB,)
    d_in = FIELDS * EMBED
    return pl.pallas_call(
        _mlp_body,
        grid=grid,
        in_specs=[
            pl.BlockSpec((BB, d_in), lambda i: (i, 0)),
            pl.BlockSpec((d_in, 64), lambda i: (0, 0)),
            pl.BlockSpec((1, 64), lambda i: (0, 0)),
            pl.BlockSpec((64, 16), lambda i: (0, 0)),
            pl.BlockSpec((1, 16), lambda i: (0, 0)),
            pl.BlockSpec((1, 16), lambda i: (0, 0)),
        ],
        out_specs=pl.BlockSpec((BB, 1), lambda i: (i, 0)),
        out_shape=jax.ShapeDtypeStruct((BATCH, 1), jnp.float32),
    )(embeds, W0, b0.reshape(1, 64), W1, b1.reshape(1, 16), W2.reshape(1, 16))


def kernel(x, table, W0, b0, W1, b1, W2):
    idx = x.reshape(NW, NCHUNK, CHUNK)
    rows = _gather(idx, table)
    embeds = rows.reshape(BATCH, FIELDS * EMBED)
    return _mlp(embeds, W0, b0, W1, b1, W2)


# speedup vs baseline: 17.7105x; 1.2121x over previous
import functools

import jax
import jax.numpy as jnp
from jax import lax
from jax.experimental import pallas as pl
from jax.experimental.pallas import tpu as pltpu
from jax.experimental.pallas import tpu_sc as plsc

VOCAB = 1000000
EMBED = 16
FIELDS = 26
BATCH = 16384

N = BATCH * FIELDS
NC = 2
NS = 16
NW = NC * NS
PER_W = N // NW
CHUNK = 1664
NCHUNK = PER_W // CHUNK


def _make_gather():
    mesh = plsc.VectorSubcoreMesh(core_axis_name="c", subcore_axis_name="s")

    @functools.partial(
        pl.kernel,
        mesh=mesh,
        out_type=jax.ShapeDtypeStruct((N, EMBED), jnp.float32),
        scratch_types=[
            pltpu.VMEM((NCHUNK, CHUNK), jnp.int32),
            pltpu.VMEM((CHUNK, EMBED), jnp.float32),
            pltpu.VMEM((CHUNK, EMBED), jnp.float32),
            pltpu.SemaphoreType.DMA,
            pltpu.SemaphoreType.DMA,
            pltpu.SemaphoreType.DMA,
            pltpu.SemaphoreType.DMA,
        ],
        compiler_params=pltpu.CompilerParams(use_tc_tiling_on_sc=False),
    )
    def gather_k(x_hbm, table_hbm, out_hbm, idx_v, buf0, buf1, g0, g1, s0, s1):
        wid = lax.axis_index("s") * NC + lax.axis_index("c")
        pltpu.sync_copy(x_hbm.at[wid], idx_v)
        base = wid * PER_W

        bufs = (buf0, buf1)
        gsems = (g0, g1)
        ssems = (s0, s1)
        gather_h = [None, None]
        store_h = [None, None]

        gather_h[0] = pltpu.async_copy(table_hbm.at[idx_v.at[0]], bufs[0],
                                       gsems[0])
        for j in range(NCHUNK):
            p = j % 2
            q = (j + 1) % 2
            if j >= 1:
                store_h[q].wait()
            if j + 1 < NCHUNK:
                gather_h[q] = pltpu.async_copy(
                    table_hbm.at[idx_v.at[j + 1]], bufs[q], gsems[q])
            gather_h[p].wait()
            store_h[p] = pltpu.async_copy(
                bufs[p], out_hbm.at[pl.ds(base + j * CHUNK, CHUNK)], ssems[p])
        store_h[(NCHUNK - 1) % 2].wait()

    return gather_k


_gather = _make_gather()


D_IN = FIELDS * EMBED
GROUP = 4
KDIM = GROUP * D_IN
NGRP = BATCH // GROUP
GB = 512


def _mlp_body(e_ref, w0_ref, b0_ref, w1_ref, b1_ref, w2_ref, o_ref):
    e = e_ref[...].reshape(GB, KDIM)
    h = jnp.dot(e, w0_ref[...], preferred_element_type=jnp.float32)
    h = jnp.maximum(h + b0_ref[...], 0.0)
    h = jnp.dot(h, w1_ref[...], preferred_element_type=jnp.float32)
    h = jnp.maximum(h + b1_ref[...], 0.0)
    logits = jnp.dot(h, w2_ref[...], preferred_element_type=jnp.float32)
    o_ref[...] = 1.0 / (1.0 + jnp.exp(-logits))


def _mlp(emb128, W0b, b0b, W1b, b1b, W2b):
    rows_per_blk = GB * KDIM // 128
    return pl.pallas_call(
        _mlp_body,
        grid=(NGRP // GB,),
        in_specs=[
            pl.BlockSpec((rows_per_blk, 128), lambda i: (i, 0)),
            pl.BlockSpec((KDIM, GROUP * 64), lambda i: (0, 0)),
            pl.BlockSpec((1, GROUP * 64), lambda i: (0, 0)),
            pl.BlockSpec((GROUP * 64, GROUP * 16), lambda i: (0, 0)),
            pl.BlockSpec((1, GROUP * 16), lambda i: (0, 0)),
            pl.BlockSpec((GROUP * 16, GROUP), lambda i: (0, 0)),
        ],
        out_specs=pl.BlockSpec((GB, GROUP), lambda i: (i, 0)),
        out_shape=jax.ShapeDtypeStruct((NGRP, GROUP), jnp.float32),
    )(emb128, W0b, b0b, W1b, b1b, W2b)


def kernel(x, table, W0, b0, W1, b1, W2):
    idx = x.reshape(NW, NCHUNK, CHUNK)
    rows = _gather(idx, table)
    emb128 = rows.reshape(N * EMBED // 128, 128)
    W0b = jnp.concatenate(
        [jnp.pad(W0, ((D_IN * p, D_IN * (GROUP - 1 - p)), (0, 0)))
         for p in range(GROUP)], axis=1)
    b0b = jnp.tile(b0, (GROUP,)).reshape(1, GROUP * 64)
    W1b = jnp.kron(jnp.eye(GROUP, dtype=jnp.float32), W1)
    b1b = jnp.tile(b1, (GROUP,)).reshape(1, GROUP * 16)
    W2b = jnp.kron(jnp.eye(GROUP, dtype=jnp.float32), W2)
    preds = _mlp(emb128, W0b, b0b, W1b, b1b, W2b)
    return preds.reshape(BATCH, 1)
